# Initial kernel scaffold; baseline (speedup 1.0000x reference)
import functools

import jax
import jax.numpy as jnp
from jax import lax
from jax.experimental import pallas as pl
from jax.experimental.pallas import tpu as pltpu
from jax.experimental.pallas import tpu_sc as plsc

N = 10000
E = 320000
D = 128

NC = 2
NS = 16
NW = NC * NS
EPW = E // NW
C = 80
NCHUNK = EPW // C
ZROWS = 125
ZITER = N // (NS * ZROWS)

_MESH = plsc.VectorSubcoreMesh(core_axis_name="c", subcore_axis_name="s")


@functools.partial(
    pl.kernel,
    mesh=_MESH,
    out_type=jax.ShapeDtypeStruct((NC, N, D), jnp.float32),
    scratch_types=[
        pltpu.VMEM((NCHUNK, C), jnp.int32),
        pltpu.VMEM((NCHUNK, C), jnp.int32),
        pltpu.VMEM((NCHUNK, C), jnp.float32),
        pltpu.VMEM((C, D), jnp.float32),
        pltpu.VMEM((ZROWS, D), jnp.float32),
        pltpu.VMEM_SHARED((N, D), jnp.float32),
        pltpu.SemaphoreType.DMA,
    ],
)
def _sc_scatter(x_hbm, src_hbm, dst_hbm, w_hbm, out_hbm,
                src_v, dst_v, w_v, rows_v, zero_v, acc_sh, sem):
    c = lax.axis_index("c")
    s = lax.axis_index("s")
    wid = s * NC + c

    def _zero_body(i, _):
        for j in range(D // 16):
            zero_v[i, pl.ds(j * 16, 16)] = jnp.zeros((16,), jnp.float32)
        return 0
    lax.fori_loop(0, ZROWS, _zero_body, 0)
    for t in range(ZITER):
        base = (s * ZITER + t) * ZROWS
        pltpu.sync_copy(zero_v, acc_sh.at[pl.ds(base, ZROWS)])
    plsc.subcore_barrier()

    pltpu.sync_copy(src_hbm.at[wid], src_v)
    pltpu.sync_copy(dst_hbm.at[wid], dst_v)
    pltpu.sync_copy(w_hbm.at[wid], w_v)

    def _chunk_body(k, _):
        pltpu.async_copy(x_hbm.at[src_v.at[k]], rows_v, sem).wait()

        def _edge_body(t, _):
            for u in range(4):
                i = t * 4 + u
                w = w_v[k, i]
                for j in range(D // 16):
                    sl = pl.ds(j * 16, 16)
                    rows_v[i, sl] = rows_v[i, sl] * w
            return 0
        lax.fori_loop(0, C // 4, _edge_body, 0)

        pltpu.sync_copy(rows_v, acc_sh.at[dst_v.at[k]], add=True)
        return 0
    lax.fori_loop(0, NCHUNK, _chunk_body, 0)

    plsc.subcore_barrier()

    for t in range(ZITER):
        base = (s * ZITER + t) * ZROWS
        pltpu.sync_copy(acc_sh.at[pl.ds(base, ZROWS)],
                        out_hbm.at[c, pl.ds(base, ZROWS)])


def _tc_body(p_ref, w_ref, b_ref, o_ref):
    p = p_ref[0] + p_ref[1]
    o_ref[...] = (
        jnp.dot(p, w_ref[...], preferred_element_type=jnp.float32)
        + b_ref[...]
    )


_TC_BLK = 1000


def _tc_matmul(partials, W, b2):
    return pl.pallas_call(
        _tc_body,
        grid=(N // _TC_BLK,),
        in_specs=[
            pl.BlockSpec((NC, _TC_BLK, D), lambda i: (0, i, 0)),
            pl.BlockSpec((D, D), lambda i: (0, 0)),
            pl.BlockSpec((1, D), lambda i: (0, 0)),
        ],
        out_specs=pl.BlockSpec((_TC_BLK, D), lambda i: (i, 0)),
        out_shape=jax.ShapeDtypeStruct((N, D), jnp.float32),
    )(partials, W, b2)


def kernel(input, edge_index, edge_weight, W, b):
    src = edge_index[1].reshape(NW, NCHUNK, C)
    dst = edge_index[0].reshape(NW, NCHUNK, C)
    w3 = edge_weight.reshape(NW, NCHUNK, C)
    partials = _sc_scatter(input, src, dst, w3)
    return _tc_matmul(partials, W, b.reshape(1, D))


# speedup vs baseline: 6.3067x; 6.3067x over previous
import functools

import jax
import jax.numpy as jnp
from jax import lax
from jax.experimental import pallas as pl
from jax.experimental.pallas import tpu as pltpu
from jax.experimental.pallas import tpu_sc as plsc

N = 10000
E = 320000
D = 128

NC = 2
NS = 16
NW = NC * NS
C = 128
NCHUNK = E // C
TPW = -(-NCHUNK // NW)
ZROWS = 80
ZITER = 8

_MESH = plsc.VectorSubcoreMesh(core_axis_name="c", subcore_axis_name="s")


@functools.partial(
    pl.kernel,
    mesh=_MESH,
    out_type=jax.ShapeDtypeStruct((NC, N, D), jnp.float32),
    scratch_types=[
        pltpu.VMEM((2, C), jnp.int32),
        pltpu.VMEM((C,), jnp.float32),
        pltpu.VMEM((C, D), jnp.float32),
        pltpu.VMEM((ZROWS, D), jnp.float32),
        pltpu.VMEM_SHARED((N, D), jnp.float32),
        pltpu.SemaphoreType.DMA,
    ],
)
def _sc_scatter(x_hbm, ei_hbm, ew_hbm, out_hbm,
                ei_v, w_v, rows_v, zero_v, acc_sh, sem):
    c = lax.axis_index("c")
    s = lax.axis_index("s")
    wid = s * NC + c

    def _zero_body(i, _):
        for j in range(D // 16):
            zero_v[i, pl.ds(j * 16, 16)] = jnp.zeros((16,), jnp.float32)
        return 0
    lax.fori_loop(0, ZROWS, _zero_body, 0)
    for t in range(ZITER):
        base = pl.multiple_of((s * ZITER + t) * ZROWS, ZROWS)

        @pl.when(base + ZROWS <= N)
        def _():
            pltpu.sync_copy(zero_v, acc_sh.at[pl.ds(base, ZROWS)])
    plsc.subcore_barrier()

    def _chunk_body(t, _):
        q = wid + t * NW

        @pl.when(q < NCHUNK)
        def _():
            e0 = pl.multiple_of(q * C, C)
            pltpu.sync_copy(ei_hbm.at[:, pl.ds(e0, C)], ei_v)
            pltpu.sync_copy(ew_hbm.at[pl.ds(e0, C)], w_v)
            pltpu.async_copy(x_hbm.at[ei_v.at[1]], rows_v, sem).wait()

            def _edge_body(g, _):
                wvec = w_v[pl.ds(g * 16, 16)]
                for u in range(16):
                    i = g * 16 + u
                    w = wvec[u]
                    for j in range(D // 16):
                        sl = pl.ds(j * 16, 16)
                        rows_v[i, sl] = rows_v[i, sl] * w
                return 0
            lax.fori_loop(0, C // 16, _edge_body, 0)

            pltpu.sync_copy(rows_v, acc_sh.at[ei_v.at[0]], add=True)
        return 0
    lax.fori_loop(0, TPW, _chunk_body, 0)

    plsc.subcore_barrier()

    for t in range(ZITER):
        base = pl.multiple_of((s * ZITER + t) * ZROWS, ZROWS)

        @pl.when(base + ZROWS <= N)
        def _():
            pltpu.sync_copy(acc_sh.at[pl.ds(base, ZROWS)],
                            out_hbm.at[c, pl.ds(base, ZROWS)])


def _tc_body(p_ref, w_ref, b_ref, o_ref):
    p = p_ref[0] + p_ref[1]
    o_ref[...] = (
        jnp.dot(p, w_ref[...], preferred_element_type=jnp.float32)
        + b_ref[...]
    )


_TC_BLK = 1000


def _tc_matmul(partials, W, b2):
    return pl.pallas_call(
        _tc_body,
        grid=(N // _TC_BLK,),
        in_specs=[
            pl.BlockSpec((NC, _TC_BLK, D), lambda i: (0, i, 0)),
            pl.BlockSpec((D, D), lambda i: (0, 0)),
            pl.BlockSpec((1, D), lambda i: (0, 0)),
        ],
        out_specs=pl.BlockSpec((_TC_BLK, D), lambda i: (i, 0)),
        out_shape=jax.ShapeDtypeStruct((N, D), jnp.float32),
    )(partials, W, b2)


def kernel(input, edge_index, edge_weight, W, b):
    partials = _sc_scatter(input, edge_index, edge_weight)
    return _tc_matmul(partials, W, b.reshape(1, D))


# speedup vs baseline: 12.0983x; 1.9183x over previous
import functools

import jax
import jax.numpy as jnp
from jax import lax
from jax.experimental import pallas as pl
from jax.experimental.pallas import tpu as pltpu
from jax.experimental.pallas import tpu_sc as plsc

N = 10000
E = 320000
D = 128

NC = 2
NS = 16
NW = NC * NS
C = 128
NCHUNK = E // C
STEPS = 80
ER = 8
RR = 2
WB = 80

_MESH = plsc.VectorSubcoreMesh(core_axis_name="c", subcore_axis_name="s")


@functools.partial(
    pl.kernel,
    mesh=_MESH,
    out_type=jax.ShapeDtypeStruct((NC, N, D), jnp.float32),
    scratch_types=[
        pltpu.VMEM((ER, 2, C), jnp.int32),
        pltpu.VMEM((ER, C), jnp.float32),
        pltpu.VMEM((RR, C, D), jnp.float32),
        pltpu.VMEM_SHARED((N, D), jnp.float32),
        pltpu.SemaphoreType.DMA((ER,)),
        pltpu.SemaphoreType.DMA((ER,)),
        pltpu.SemaphoreType.DMA((RR,)),
        pltpu.SemaphoreType.DMA((RR,)),
    ],
)
def _sc_scatter(x_hbm, ei_hbm, ew_hbm, out_hbm,
                ei_v, w_v, rows_v, acc_sh,
                sem_e, sem_w, sem_g, sem_s):
    c = lax.axis_index("c")
    s = lax.axis_index("s")
    wid = s * NC + c

    def chunk_off(i):
        q = jnp.minimum(wid + i * NW, NCHUNK - 1)
        return pl.multiple_of(q * C, C)

    def start_edges(i, slot):
        e0 = chunk_off(i)
        pltpu.async_copy(ei_hbm.at[:, pl.ds(e0, C)], ei_v.at[slot],
                         sem_e.at[slot])
        pltpu.async_copy(ew_hbm.at[pl.ds(e0, C)], w_v.at[slot],
                         sem_w.at[slot])

    def wait_edges_ei(i, slot):
        e0 = chunk_off(i)
        pltpu.make_async_copy(ei_hbm.at[:, pl.ds(e0, C)], ei_v.at[slot],
                              sem_e.at[slot]).wait()

    def wait_edges_ew(i, slot):
        e0 = chunk_off(i)
        pltpu.make_async_copy(ew_hbm.at[pl.ds(e0, C)], w_v.at[slot],
                              sem_w.at[slot]).wait()

    def start_gather(eslot, rslot):
        pltpu.async_copy(x_hbm.at[ei_v.at[eslot, 1]], rows_v.at[rslot],
                         sem_g.at[rslot])

    def wait_gather(eslot, rslot):
        pltpu.make_async_copy(x_hbm.at[ei_v.at[eslot, 1]], rows_v.at[rslot],
                              sem_g.at[rslot]).wait()

    def start_scatter(eslot, rslot):
        pltpu.async_copy(rows_v.at[rslot], acc_sh.at[ei_v.at[eslot, 0]],
                         sem_s.at[rslot], add=True)

    def wait_scatter(eslot, rslot):
        pltpu.make_async_copy(rows_v.at[rslot], acc_sh.at[ei_v.at[eslot, 0]],
                              sem_s.at[rslot]).wait()

    def _zero_body(i, _):
        for j in range(D // 16):
            rows_v[0, i, pl.ds(j * 16, 16)] = jnp.zeros((16,), jnp.float32)
        return 0
    lax.fori_loop(0, C, _zero_body, 0)
    for t in range(WB // NS):
        idx = s * (WB // NS) + t
        base = pl.multiple_of(idx * C, C)

        @pl.when(base + C <= N)
        def _():
            pltpu.sync_copy(rows_v.at[0], acc_sh.at[pl.ds(base, C)])

        @pl.when(idx == (N // C))
        def _():
            pltpu.sync_copy(rows_v.at[0, pl.ds(0, N % C)],
                            acc_sh.at[pl.ds(N - N % C, N % C)])

    for j in range(4):
        start_edges(j, j)
    wait_edges_ei(0, 0)
    start_gather(0, 0)

    plsc.subcore_barrier()

    def _step_body(step, _):
        for b in range(ER):
            i = step * ER + b
            rb = b % RR
            wait_gather(b, rb)
            if b == 0:
                @pl.when(step > 0)
                def _():
                    wait_scatter((b + ER - 1) % ER, (rb + 1) % RR)
            else:
                wait_scatter((b + ER - 1) % ER, (rb + 1) % RR)
            if b == ER - 1:
                @pl.when(step < STEPS // ER - 1)
                def _():
                    wait_edges_ei(i + 1, (b + 1) % ER)
                    start_gather((b + 1) % ER, (rb + 1) % RR)
            else:
                wait_edges_ei(i + 1, (b + 1) % ER)
                start_gather((b + 1) % ER, (rb + 1) % RR)
            if b >= 4:
                @pl.when(step < STEPS // ER - 1)
                def _():
                    start_edges(i + 4, (b + 4) % ER)
            else:
                start_edges(i + 4, (b + 4) % ER)
            wait_edges_ew(i, b)
            vf = jnp.where(wid + i * NW < NCHUNK, 1.0, 0.0)

            def _edge_body(g, _):
                wvec = w_v[b, pl.ds(g * 16, 16)] * vf
                for u in range(16):
                    e = g * 16 + u
                    w = wvec[u]
                    for j in range(D // 16):
                        sl = pl.ds(j * 16, 16)
                        rows_v[rb, e, sl] = rows_v[rb, e, sl] * w
                return 0
            lax.fori_loop(0, C // 16, _edge_body, 0)
            start_scatter(b, rb)
        return 0
    lax.fori_loop(0, STEPS // ER, _step_body, 0)

    wait_scatter((STEPS - 1) % ER, (STEPS - 1) % RR)

    plsc.subcore_barrier()

    for t in range(WB // NS):
        idx = s * (WB // NS) + t
        base = pl.multiple_of(idx * C, C)

        @pl.when(base + C <= N)
        def _():
            pltpu.sync_copy(acc_sh.at[pl.ds(base, C)],
                            out_hbm.at[c, pl.ds(base, C)])

        @pl.when(idx == (N // C))
        def _():
            pltpu.sync_copy(acc_sh.at[pl.ds(N - N % C, N % C)],
                            out_hbm.at[c, pl.ds(N - N % C, N % C)])


def _tc_body(p_ref, w_ref, b_ref, o_ref):
    p = p_ref[0] + p_ref[1]
    o_ref[...] = (
        jnp.dot(p, w_ref[...], preferred_element_type=jnp.float32)
        + b_ref[...]
    )


_TC_BLK = 1000


def _tc_matmul(partials, W, b2):
    return pl.pallas_call(
        _tc_body,
        grid=(N // _TC_BLK,),
        in_specs=[
            pl.BlockSpec((NC, _TC_BLK, D), lambda i: (0, i, 0)),
            pl.BlockSpec((D, D), lambda i: (0, 0)),
            pl.BlockSpec((1, D), lambda i: (0, 0)),
        ],
        out_specs=pl.BlockSpec((_TC_BLK, D), lambda i: (i, 0)),
        out_shape=jax.ShapeDtypeStruct((N, D), jnp.float32),
    )(partials, W, b2)


def kernel(input, edge_index, edge_weight, W, b):
    partials = _sc_scatter(input, edge_index, edge_weight)
    return _tc_matmul(partials, W, b.reshape(1, D))
